# baseline (speedup 1.0000x reference)
import jax
import jax.numpy as jnp
from jax.experimental import pallas as pl
from jax.experimental.pallas import tpu as pltpu

N_ROWS = 32768
D = 768
NA = 18
NAPAD = 24
KSEL = 50
KPAD = 64
NBLK = 8
BLK = N_ROWS // NBLK


def _body(x_hbm, x_ref, w_ref, scores_out, spans_out, cls_out, emb_out,
          max_scr, key_scr, idx_smem, sem):
    g = pl.program_id(0)
    xb = x_ref[...].astype(jnp.bfloat16)
    st = jax.lax.dot_general(w_ref[...], xb, (((1,), (1,)), ((), ())),
                             preferred_element_type=jnp.float32)
    row = jax.lax.broadcasted_iota(jnp.int32, (NAPAD, 1), 0)
    m = jnp.max(st, axis=0)
    cls = jnp.min(jnp.where(st == m[None, :], row, NAPAD),
                  axis=0).astype(jnp.int32)
    col = jax.lax.iota(jnp.int32, BLK)
    max_scr[g, :] = m
    key_scr[g, :] = (g * BLK + col) * 32 + cls

    @pl.when(g == NBLK - 1)
    def _():
        lane = jax.lax.broadcasted_iota(jnp.int32, (1, 128), 1)
        nv = BLK // 128

        def merge(v1, k1, v2, k2):
            t = (v1 > v2) | ((v1 == v2) & (k1 < k2))
            return jnp.where(t, v1, v2), jnp.where(t, k1, k2)

        def slot_reduce(a3, k3):
            pairs = [(a3[:, j], k3[:, j]) for j in range(nv)]
            while len(pairs) > 1:
                nxt = [(merge(*pairs[2 * t], *pairs[2 * t + 1]))
                       for t in range(len(pairs) // 2)]
                if len(pairs) % 2:
                    nxt.append(pairs[-1])
                pairs = nxt
            return pairs[0]

        a3 = max_scr[...].reshape(NBLK, nv, 128)
        k3 = key_scr[...].reshape(NBLK, nv, 128)
        accv = jnp.zeros((1, 128), jnp.float32)
        acck = jnp.zeros((1, 128), jnp.int32)
        for i in range(KSEL):
            rv, rk = slot_reduce(a3, k3)
            for s in (64, 32, 16, 8, 4, 2, 1):
                rv, rk = merge(rv, rk, pltpu.roll(rv, s, 1),
                               pltpu.roll(rk, s, 1))
            for s in (4, 2, 1):
                rv, rk = merge(rv, rk, pltpu.roll(rv, s, 0),
                               pltpu.roll(rk, s, 0))
            oh = lane == i
            accv = jnp.where(oh, rv[0:1], accv)
            acck = jnp.where(oh, rk[0:1], acck)
            a3 = jnp.where(k3 == rk[:, None, :], -jnp.inf, a3)
        scores_out[...] = accv[0]
        spans_out[...] = jax.lax.shift_right_logical(acck[0], 5)
        cls_out[...] = jax.lax.bitwise_and(acck[0], 31)
        cp = pltpu.make_async_copy(spans_out, idx_smem, sem)
        cp.start()
        cp.wait()
        for i in range(KSEL):
            pltpu.make_async_copy(
                x_hbm.at[pl.ds(idx_smem[i], 1), :],
                emb_out.at[pl.ds(i, 1), :], sem).start()
        for i in range(KSEL):
            pltpu.make_async_copy(
                x_hbm.at[pl.ds(0, 1), :],
                emb_out.at[pl.ds(i, 1), :], sem).wait()


def kernel(embs, entity_anchor, k):
    del k
    w_pad = jnp.concatenate(
        [entity_anchor,
         jnp.broadcast_to(entity_anchor[:1], (NAPAD - NA, D))],
        axis=0).astype(jnp.bfloat16)
    scores, spans, cls, emb = pl.pallas_call(
        _body,
        grid=(NBLK,),
        in_specs=[
            pl.BlockSpec(memory_space=pl.ANY),
            pl.BlockSpec((BLK, D), lambda g: (g, 0)),
            pl.BlockSpec((NAPAD, D), lambda g: (0, 0)),
        ],
        out_specs=[
            pl.BlockSpec((128,), lambda g: (0,)),
            pl.BlockSpec((128,), lambda g: (0,)),
            pl.BlockSpec((128,), lambda g: (0,)),
            pl.BlockSpec((KPAD, D), lambda g: (0, 0)),
        ],
        out_shape=[
            jax.ShapeDtypeStruct((128,), jnp.float32),
            jax.ShapeDtypeStruct((128,), jnp.int32),
            jax.ShapeDtypeStruct((128,), jnp.int32),
            jax.ShapeDtypeStruct((KPAD, D), jnp.float32),
        ],
        scratch_shapes=[
            pltpu.VMEM((NBLK, BLK), jnp.float32),
            pltpu.VMEM((NBLK, BLK), jnp.int32),
            pltpu.SMEM((128,), jnp.int32),
            pltpu.SemaphoreType.DMA,
        ],
        compiler_params=pltpu.CompilerParams(
            dimension_semantics=("arbitrary",)),
    )(embs, embs, w_pad)
    return scores[:KSEL], spans[:KSEL], cls[:KSEL], emb[:KSEL]
